# Initial kernel scaffold; baseline (speedup 1.0000x reference)
import jax
import jax.numpy as jnp
from jax.experimental import pallas as pl


def kernel(x, edge_index, Wl1, bl1, Wr1, Wl2, bl2, Wr2, Wo, bo):
    raise NotImplementedError("write your pallas kernel here")


# speedup vs baseline: 3.3227x; 3.3227x over previous
import functools

import jax
import jax.numpy as jnp
from jax import lax
from jax.experimental import pallas as pl
from jax.experimental.pallas import tpu as pltpu
from jax.experimental.pallas import tpu_sc as plsc

N_NODES = 10000
N_EDGES = 320000
D = 128

NC = 2
NS = 16
NW = NC * NS
CHUNK = 128
NCH = 80
EPW = NCH * CHUNK
E_PAD = NW * EPW
N_PAD = 10240
RPS = N_PAD // NS
DUMP_ROW = N_PAD - 1

_mesh = plsc.VectorSubcoreMesh(core_axis_name="c", subcore_axis_name="s")


def _seg_sum_body(with_counts, *refs):
    if with_counts:
        (y_hbm, src_hbm, dst_hbm, z2d_hbm, z1d_hbm, ones_hbm,
         out_hbm, cnt_hbm,
         src_vm, dst_vm, rows_v, ones_v,
         acc, cntacc, sem) = refs
    else:
        (y_hbm, src_hbm, dst_hbm, z2d_hbm,
         out_hbm,
         src_vm, dst_vm, rows_v,
         acc, sem) = refs

    cid = lax.axis_index("c")
    sid = lax.axis_index("s")
    wid = cid * NS + sid

    stripe = pl.ds(sid * RPS, RPS)
    pltpu.sync_copy(z2d_hbm.at[stripe], acc.at[stripe])
    if with_counts:
        pltpu.sync_copy(z1d_hbm.at[stripe], cntacc.at[stripe])
        pltpu.sync_copy(ones_hbm, ones_v)
    pltpu.sync_copy(src_hbm.at[wid], src_vm)
    pltpu.sync_copy(dst_hbm.at[wid], dst_vm)
    plsc.subcore_barrier()

    def chunk(j, carry):
        pltpu.async_copy(y_hbm.at[src_vm.at[j]], rows_v, sem).wait()
        pltpu.sync_copy(rows_v, acc.at[dst_vm.at[j]], add=True)
        if with_counts:
            pltpu.sync_copy(ones_v, cntacc.at[dst_vm.at[j]], add=True)
        return carry

    lax.fori_loop(0, NCH, chunk, 0)

    plsc.subcore_barrier()

    pltpu.sync_copy(acc.at[stripe], out_hbm.at[cid, stripe])
    if with_counts:
        pltpu.sync_copy(cntacc.at[stripe], cnt_hbm.at[cid, stripe])


_seg_sum_counts = pl.kernel(
    functools.partial(_seg_sum_body, True),
    out_type=(jax.ShapeDtypeStruct((NC, N_PAD, D), jnp.float32),
              jax.ShapeDtypeStruct((NC, N_PAD), jnp.float32)),
    mesh=_mesh,
    scratch_types=[
        pltpu.VMEM((NCH, CHUNK), jnp.int32),
        pltpu.VMEM((NCH, CHUNK), jnp.int32),
        pltpu.VMEM((CHUNK, D), jnp.float32),
        pltpu.VMEM((CHUNK,), jnp.float32),
        pltpu.VMEM_SHARED((N_PAD, D), jnp.float32),
        pltpu.VMEM_SHARED((N_PAD,), jnp.float32),
        pltpu.SemaphoreType.DMA,
    ],
)

_seg_sum_plain = pl.kernel(
    functools.partial(_seg_sum_body, False),
    out_type=jax.ShapeDtypeStruct((NC, N_PAD, D), jnp.float32),
    mesh=_mesh,
    scratch_types=[
        pltpu.VMEM((NCH, CHUNK), jnp.int32),
        pltpu.VMEM((NCH, CHUNK), jnp.int32),
        pltpu.VMEM((CHUNK, D), jnp.float32),
        pltpu.VMEM_SHARED((N_PAD, D), jnp.float32),
        pltpu.SemaphoreType.DMA,
    ],
)


BLK = 2000
GRID = N_NODES // BLK


def _mm2_body(x_ref, wa_ref, wb_ref, ya_ref, yb_ref):
    x = x_ref[...]
    ya_ref[...] = jnp.dot(x, wa_ref[...], preferred_element_type=jnp.float32)
    yb_ref[...] = jnp.dot(x, wb_ref[...], preferred_element_type=jnp.float32)


_mm2 = pl.pallas_call(
    _mm2_body,
    grid=(GRID,),
    in_specs=[
        pl.BlockSpec((BLK, D), lambda i: (i, 0)),
        pl.BlockSpec((D, D), lambda i: (0, 0)),
        pl.BlockSpec((D, D), lambda i: (0, 0)),
    ],
    out_specs=[
        pl.BlockSpec((BLK, D), lambda i: (i, 0)),
        pl.BlockSpec((BLK, D), lambda i: (i, 0)),
    ],
    out_shape=[jax.ShapeDtypeStruct((N_NODES, D), jnp.float32),
               jax.ShapeDtypeStruct((N_NODES, D), jnp.float32)],
)


def _mid_body(s0_ref, s1_ref, c0_ref, c1_ref, bl_ref, r_ref,
              wa_ref, wb_ref, ya_ref, yb_ref):
    cnt = jnp.maximum(c0_ref[0] + c1_ref[0], 1.0)
    h = (s0_ref[0] + s1_ref[0]) / cnt + bl_ref[...] + r_ref[...]
    h = jnp.maximum(h, 0.0)
    ya_ref[...] = jnp.dot(h, wa_ref[...], preferred_element_type=jnp.float32)
    yb_ref[...] = jnp.dot(h, wb_ref[...], preferred_element_type=jnp.float32)


_mid = pl.pallas_call(
    _mid_body,
    grid=(GRID,),
    in_specs=[
        pl.BlockSpec((1, BLK, D), lambda i: (0, i, 0)),
        pl.BlockSpec((1, BLK, D), lambda i: (1, i, 0)),
        pl.BlockSpec((1, BLK, 1), lambda i: (0, i, 0)),
        pl.BlockSpec((1, BLK, 1), lambda i: (1, i, 0)),
        pl.BlockSpec((1, D), lambda i: (0, 0)),
        pl.BlockSpec((BLK, D), lambda i: (i, 0)),
        pl.BlockSpec((D, D), lambda i: (0, 0)),
        pl.BlockSpec((D, D), lambda i: (0, 0)),
    ],
    out_specs=[
        pl.BlockSpec((BLK, D), lambda i: (i, 0)),
        pl.BlockSpec((BLK, D), lambda i: (i, 0)),
    ],
    out_shape=[jax.ShapeDtypeStruct((N_NODES, D), jnp.float32),
               jax.ShapeDtypeStruct((N_NODES, D), jnp.float32)],
)


def _fin_body(s0_ref, s1_ref, c0_ref, c1_ref, bl_ref, r_ref,
              wo_ref, bo_ref, o_ref):
    cnt = jnp.maximum(c0_ref[0] + c1_ref[0], 1.0)
    h = (s0_ref[0] + s1_ref[0]) / cnt + bl_ref[...] + r_ref[...]
    h = jnp.maximum(h, 0.0)
    o_ref[...] = jnp.dot(h, wo_ref[...], preferred_element_type=jnp.float32) + bo_ref[...]


_fin = pl.pallas_call(
    _fin_body,
    grid=(GRID,),
    in_specs=[
        pl.BlockSpec((1, BLK, D), lambda i: (0, i, 0)),
        pl.BlockSpec((1, BLK, D), lambda i: (1, i, 0)),
        pl.BlockSpec((1, BLK, 1), lambda i: (0, i, 0)),
        pl.BlockSpec((1, BLK, 1), lambda i: (1, i, 0)),
        pl.BlockSpec((1, D), lambda i: (0, 0)),
        pl.BlockSpec((BLK, D), lambda i: (i, 0)),
        pl.BlockSpec((D, 1), lambda i: (0, 0)),
        pl.BlockSpec((1, 1), lambda i: (0, 0)),
    ],
    out_specs=pl.BlockSpec((BLK, 1), lambda i: (i, 0)),
    out_shape=jax.ShapeDtypeStruct((N_NODES, 1), jnp.float32),
)


def kernel(x, edge_index, Wl1, bl1, Wr1, Wl2, bl2, Wr2, Wo, bo):
    src = edge_index[0].astype(jnp.int32)
    dst = edge_index[1].astype(jnp.int32)
    npad = E_PAD - N_EDGES
    srcp = jnp.concatenate([src, jnp.zeros((npad,), jnp.int32)]).reshape(NW, NCH, CHUNK)
    dstp = jnp.concatenate([dst, jnp.full((npad,), DUMP_ROW, jnp.int32)]).reshape(NW, NCH, CHUNK)
    z2d = jnp.zeros((N_PAD, D), jnp.float32)
    z1d = jnp.zeros((N_PAD,), jnp.float32)
    ones = jnp.ones((CHUNK,), jnp.float32)

    y1, r1 = _mm2(x, Wl1, Wr1)
    s1, cnt = _seg_sum_counts(y1, srcp, dstp, z2d, z1d, ones)
    cnt3 = cnt.reshape(NC, N_PAD, 1)
    y2, r2 = _mid(s1, s1, cnt3, cnt3, bl1.reshape(1, D), r1, Wl2, Wr2)
    s2 = _seg_sum_plain(y2, srcp, dstp, z2d)
    out = _fin(s2, s2, cnt3, cnt3, bl2.reshape(1, D), r2,
               Wo, bo.reshape(1, 1))
    return out.reshape(N_NODES)
